# baseline (speedup 1.0000x reference)
import jax
import jax.numpy as jnp
from jax import lax
from jax.experimental import pallas as pl
from jax.experimental.pallas import tpu as pltpu
from jax.experimental.pallas import tpu_sc as plsc

_IN = 4096
_OUT = 4096
_BATCH = 1024
_NV = _BATCH // 16
_K = 8
_NB = 3
_NCH = 384
_S = _K * _NCH
_TILES = 32
_OPT = _OUT // _TILES
_NC = 2


def _prep(in_idx, zmap):
    out_n, L = zmap.shape
    n_params = in_idx.shape[0]
    zm = zmap.astype(jnp.int32)
    valid = zm < n_params
    zsafe = jnp.where(valid, zm, 0)
    cnt = valid.sum(axis=1, dtype=jnp.int32)
    cnt2 = ((cnt + 1) // 2) * 2
    off = jnp.concatenate(
        [jnp.zeros(1, jnp.int32), jnp.cumsum(cnt2, dtype=jnp.int32)])
    o = jnp.arange(out_n, dtype=jnp.int32)
    tile = o // _OPT
    local = off[:-1] - off[tile * _OPT]
    dump = _TILES * _S
    pos = local[:, None] + jnp.arange(L, dtype=jnp.int32)[None, :]
    dest = jnp.where(valid & (pos < _S), tile[:, None] * _S + pos, dump)
    perm = jnp.full(dump + 1, n_params, jnp.int32).at[dest].set(zsafe)
    lastpos = local + cnt2 - 1
    last = jnp.where((cnt > 0) & (lastpos < _S), tile * _S + lastpos, dump)
    e_ts = jnp.full(dump + 1, -1, jnp.int32).at[last].set(o % _OPT)
    return (perm[:dump].reshape(_TILES, _S),
            e_ts[:dump].reshape(_TILES, _S))


def _lane(v, i):
    return lax.squeeze(lax.slice_in_dim(v, i, i + 1), (0,))


def _make_body(n_params):
    def _body(xt, perm_ts, e_ts, w_hbm, gf_hbm, out,
              wbuf, pgbuf, ebuf, accbuf, stage, tab, ring,
              sem0, sem1, sem2, fsem):
        t = lax.axis_index("s") * _NC + lax.axis_index("c")
        sems = (sem0, sem1, sem2)
        zeros16 = jnp.zeros((16,), jnp.float32)

        half = ((n_params + 1) // 2 + 7) // 8 * 8
        rest = n_params - half
        tab[pl.ds(half, 16)] = zeros16
        pltpu.sync_copy(perm_ts.at[t], pgbuf)

        def half_gather(lo, write):
            def body(j, carry):
                sl = pl.ds(j * 16, 16)
                idxv = pgbuf[sl] - lo
                m = (idxv >= 0) & (idxv < half)
                g = plsc.load_gather(tab, [jnp.where(m, idxv, 0)], mask=m)
                write(sl, m, g)
                return carry

            lax.fori_loop(0, _S // 16, body, 0)

        pltpu.sync_copy(w_hbm.at[pl.ds(0, half)], tab.at[pl.ds(0, half)])
        half_gather(0, lambda sl, m, g: wbuf.__setitem__(
            sl, jnp.where(m, g, 0.0)))
        pltpu.sync_copy(w_hbm.at[pl.ds(half, rest)], tab.at[pl.ds(0, rest)])
        half_gather(half, lambda sl, m, g: wbuf.__setitem__(
            sl, jnp.where(m, g, wbuf[sl])))
        pltpu.sync_copy(gf_hbm.at[pl.ds(0, half)], tab.at[pl.ds(0, half)])
        half_gather(0, lambda sl, m, g: ebuf.__setitem__(
            sl, jnp.where(m, plsc.bitcast(g, jnp.int32), 0)))
        pltpu.sync_copy(gf_hbm.at[pl.ds(half, rest)], tab.at[pl.ds(0, rest)])
        half_gather(half, lambda sl, m, g: pgbuf.__setitem__(
            sl, jnp.where(m, plsc.bitcast(g, jnp.int32), ebuf[sl])))
        pltpu.sync_copy(e_ts.at[t], ebuf.at[pl.ds(0, _S)])

        def zero_acc(j, carry):
            accbuf[pl.ds(j * 16, 16)] = zeros16
            return carry

        lax.fori_loop(0, _NV, zero_acc, 0)

        def start_gather(c, b):
            pltpu.async_copy(xt.at[pgbuf.at[pl.ds(c * _K, _K)]], ring.at[b],
                             sems[b])

        def wait_gather(b):
            pltpu.make_async_copy(xt.at[pgbuf.at[pl.ds(0, _K)]], ring.at[b],
                                  sems[b]).wait()

        def wait_flush():
            pltpu.make_async_copy(stage, out.at[t * _OPT], fsem).wait()

        def chunk_compute(c, rb):
            wv = wbuf[pl.ds(c * _K, 16)]
            ev = ebuf[pl.ds(c * _K, 16)]
            for g in range(_K // 2):
                w0 = jnp.full((16,), _lane(wv, 2 * g), jnp.float32)
                w1 = jnp.full((16,), _lane(wv, 2 * g + 1), jnp.float32)
                e_s = _lane(ev, 2 * g + 1)
                for k in range(_NV):
                    sl = pl.ds(16 * k, 16)
                    plsc.addupdate(accbuf.at[sl],
                                   w0 * rb[2 * g, sl] + w1 * rb[2 * g + 1, sl])

                @pl.when(e_s >= 0)
                def _():
                    wait_flush()

                    def mv(j, carry):
                        sl = pl.ds(j * 16, 16)
                        stage[sl] = accbuf[sl]
                        accbuf[sl] = zeros16
                        return carry

                    lax.fori_loop(0, _NV, mv, 0)
                    pltpu.async_copy(stage, out.at[t * _OPT + e_s], fsem)

        pltpu.async_copy(stage, out.at[t * _OPT], fsem)

        for b in range(_NB):
            start_gather(jnp.int32(b), b)

        def outer_body(g2, carry):
            for b in range(_NB):
                c = g2 * _NB + b
                wait_gather(b)
                chunk_compute(c, ring.at[b])

                @pl.when(c + _NB < _NCH)
                def _():
                    start_gather(c + _NB, b)
            return carry

        lax.fori_loop(0, _NCH // _NB, outer_body, 0)
        wait_flush()

    return _body


def _sc_call(xt, perm_ts, e_ts, w_hbm, gf_hbm):
    n_params = w_hbm.shape[0]
    mesh = plsc.VectorSubcoreMesh(core_axis_name="c", subcore_axis_name="s")
    kern = pl.kernel(
        _make_body(n_params),
        out_type=jax.ShapeDtypeStruct((_OUT, _BATCH), jnp.float32),
        mesh=mesh,
        compiler_params=pltpu.CompilerParams(needs_layout_passes=False),
        scratch_types=[
            pltpu.VMEM((_S + 16,), jnp.float32),
            pltpu.VMEM((_S,), jnp.int32),
            pltpu.VMEM((_S + 16,), jnp.int32),
            pltpu.VMEM((_BATCH,), jnp.float32),
            pltpu.VMEM((_BATCH,), jnp.float32),
            pltpu.VMEM((((n_params + 1) // 2 + 7) // 8 * 8 + 16,),
                       jnp.float32),
            pltpu.VMEM((_NB, _K, _BATCH), jnp.float32),
            pltpu.SemaphoreType.DMA,
            pltpu.SemaphoreType.DMA,
            pltpu.SemaphoreType.DMA,
            pltpu.SemaphoreType.DMA,
        ],
    )
    return kern(xt, perm_ts, e_ts, w_hbm, gf_hbm)


def kernel(x, weight, bias, in_idx, zmap):
    perm_ts, e_ts = _prep(in_idx, zmap)
    gf = lax.bitcast_convert_type(in_idx.astype(jnp.int32), jnp.float32)
    out_t = _sc_call(x.T, perm_ts, e_ts, weight, gf)
    return out_t.T + bias[None, :]


# speedup vs baseline: 1.5849x; 1.5849x over previous
import jax
import jax.numpy as jnp
from jax import lax
from jax.experimental import pallas as pl
from jax.experimental.pallas import tpu as pltpu
from jax.experimental.pallas import tpu_sc as plsc

_IN = 4096
_OUT = 4096
_BATCH = 1024
_NV = _BATCH // 16
_K = 8
_NB = 3
_NCH = 384
_S = _K * _NCH
_TILES = 32
_OPT = _OUT // _TILES
_NC = 2


def _prep(in_idx, zmap):
    out_n, L = zmap.shape
    n_params = in_idx.shape[0]
    zm = zmap.astype(jnp.int32)
    valid = zm < n_params
    zsafe = jnp.where(valid, zm, 0)
    cnt = valid.sum(axis=1, dtype=jnp.int32)
    cnt2 = ((cnt + 1) // 2) * 2
    off = jnp.concatenate(
        [jnp.zeros(1, jnp.int32), jnp.cumsum(cnt2, dtype=jnp.int32)])
    o = jnp.arange(out_n, dtype=jnp.int32)
    tile = o // _OPT
    local = off[:-1] - off[tile * _OPT]
    dump = _TILES * _S
    pos = local[:, None] + jnp.arange(L, dtype=jnp.int32)[None, :]
    dest = jnp.where(valid & (pos < _S), tile[:, None] * _S + pos, dump)
    perm = jnp.full(dump + 1, n_params, jnp.int32).at[dest].set(zsafe)
    lastpos = local + cnt2 - 1
    last = jnp.where((cnt > 0) & (lastpos < _S), tile * _S + lastpos, dump)
    e_ts = jnp.full(dump + 1, -1, jnp.int32).at[last].set(o % _OPT)
    return (perm[:dump].reshape(_TILES, _S),
            e_ts[:dump].reshape(_TILES, _S))


def _lane(v, i):
    return lax.squeeze(lax.slice_in_dim(v, i, i + 1), (0,))


def _make_body(n_params):
    def _body(xt, perm_ts, e_ts, w_hbm, gf_hbm, out,
              wbuf, pgbuf, ebuf, accbuf, stage, tab, ring,
              sem0, sem1, sem2, fsem):
        t = lax.axis_index("s") * _NC + lax.axis_index("c")
        sems = (sem0, sem1, sem2)
        zeros16 = jnp.zeros((16,), jnp.float32)

        half = ((n_params + 1) // 2 + 7) // 8 * 8
        rest = n_params - half
        tab[pl.ds(half, 16)] = zeros16
        pltpu.sync_copy(perm_ts.at[t], pgbuf)

        def half_gather(lo, write):
            def body(j, carry):
                sl = pl.ds(j * 16, 16)
                idxv = pgbuf[sl] - lo
                m = (idxv >= 0) & (idxv < half)
                g = plsc.load_gather(tab, [jnp.where(m, idxv, 0)], mask=m)
                write(sl, m, g)
                return carry

            lax.fori_loop(0, _S // 16, body, 0)

        pltpu.sync_copy(w_hbm.at[pl.ds(0, half)], tab.at[pl.ds(0, half)])
        half_gather(0, lambda sl, m, g: wbuf.__setitem__(
            sl, jnp.where(m, g, 0.0)))
        pltpu.sync_copy(w_hbm.at[pl.ds(half, rest)], tab.at[pl.ds(0, rest)])
        half_gather(half, lambda sl, m, g: wbuf.__setitem__(
            sl, jnp.where(m, g, wbuf[sl])))
        pltpu.sync_copy(gf_hbm.at[pl.ds(0, half)], tab.at[pl.ds(0, half)])
        half_gather(0, lambda sl, m, g: ebuf.__setitem__(
            sl, jnp.where(m, plsc.bitcast(g, jnp.int32), 0)))
        pltpu.sync_copy(gf_hbm.at[pl.ds(half, rest)], tab.at[pl.ds(0, rest)])
        half_gather(half, lambda sl, m, g: pgbuf.__setitem__(
            sl, jnp.where(m, plsc.bitcast(g, jnp.int32), ebuf[sl])))
        pltpu.sync_copy(e_ts.at[t], ebuf.at[pl.ds(0, _S)])

        def zero_acc(j, carry):
            accbuf[pl.ds(j * 16, 16)] = zeros16
            return carry

        lax.fori_loop(0, _NV, zero_acc, 0)

        def start_gather(c, b):
            pltpu.async_copy(xt.at[pgbuf.at[pl.ds(c * _K, _K)]], ring.at[b],
                             sems[b])

        def wait_gather(b):
            pltpu.make_async_copy(xt.at[pgbuf.at[pl.ds(0, _K)]], ring.at[b],
                                  sems[b]).wait()

        def wait_flush():
            pltpu.make_async_copy(stage, out.at[t * _OPT], fsem).wait()

        def chunk_compute(c, rb):
            wv = wbuf[pl.ds(c * _K, 16)]
            ev = ebuf[pl.ds(c * _K, 16)]
            ws = [jnp.full((16,), _lane(wv, j), jnp.float32)
                  for j in range(_K)]
            es = [_lane(ev, 2 * j + 1) for j in range(_K // 2)]
            for qs in range(4):
                qoff = qs * 256
                acc = [accbuf[pl.ds(qoff + 16 * k, 16)] for k in range(16)]
                for j in range(_K):
                    acc = [acc[k] + ws[j] * rb[j, pl.ds(qoff + 16 * k, 16)]
                           for k in range(16)]
                    if j % 2 == 1:
                        e_s = es[j // 2]
                        flush = e_s >= 0

                        @pl.when(flush)
                        def _(acc=acc, e_s=e_s, qs=qs, qoff=qoff):
                            for k in range(16):
                                stage[pl.ds(qoff + 16 * k, 16)] = acc[k]
                            if qs == 3:
                                pltpu.async_copy(
                                    stage, out.at[t * _OPT + e_s], fsem)
                                wait_flush()

                        acc = [jnp.where(flush, 0.0, acc[k])
                               for k in range(16)]
                for k in range(16):
                    accbuf[pl.ds(qoff + 16 * k, 16)] = acc[k]

        for b in range(_NB):
            start_gather(jnp.int32(b), b)

        def outer_body(g2, carry):
            for b in range(_NB):
                c = g2 * _NB + b
                wait_gather(b)
                chunk_compute(c, ring.at[b])

                @pl.when(c + _NB < _NCH)
                def _():
                    start_gather(c + _NB, b)
            return carry

        lax.fori_loop(0, _NCH // _NB, outer_body, 0)

    return _body


def _sc_call(xt, perm_ts, e_ts, w_hbm, gf_hbm):
    n_params = w_hbm.shape[0]
    mesh = plsc.VectorSubcoreMesh(core_axis_name="c", subcore_axis_name="s")
    kern = pl.kernel(
        _make_body(n_params),
        out_type=jax.ShapeDtypeStruct((_OUT, _BATCH), jnp.float32),
        mesh=mesh,
        compiler_params=pltpu.CompilerParams(needs_layout_passes=False),
        scratch_types=[
            pltpu.VMEM((_S + 16,), jnp.float32),
            pltpu.VMEM((_S,), jnp.int32),
            pltpu.VMEM((_S + 16,), jnp.int32),
            pltpu.VMEM((_BATCH,), jnp.float32),
            pltpu.VMEM((_BATCH,), jnp.float32),
            pltpu.VMEM((((n_params + 1) // 2 + 7) // 8 * 8 + 16,),
                       jnp.float32),
            pltpu.VMEM((_NB, _K, _BATCH), jnp.float32),
            pltpu.SemaphoreType.DMA,
            pltpu.SemaphoreType.DMA,
            pltpu.SemaphoreType.DMA,
            pltpu.SemaphoreType.DMA,
        ],
    )
    return kern(xt, perm_ts, e_ts, w_hbm, gf_hbm)


def kernel(x, weight, bias, in_idx, zmap):
    perm_ts, e_ts = _prep(in_idx, zmap)
    gf = lax.bitcast_convert_type(in_idx.astype(jnp.int32), jnp.float32)
    out_t = _sc_call(x.T, perm_ts, e_ts, weight, gf)
    return out_t.T + bias[None, :]
